# baseline (speedup 1.0000x reference)
import functools

import jax
import jax.numpy as jnp
from jax import lax
from jax.experimental import pallas as pl
from jax.experimental.pallas import tpu as pltpu
from jax.experimental.pallas import tpu_sc as plsc

TM = 256
HK = 512


def _routing_metadata(rand_maps, T, E, tm):
    nt = T // tm
    W = nt + E - 1
    perm = jnp.argsort(rand_maps, stable=True).astype(jnp.int32)
    counts = jnp.bincount(rand_maps, length=E)
    ends = jnp.cumsum(counts)
    starts = ends - counts
    first_tile = starts // tm
    last_tile = (ends + tm - 1) // tm
    npairs = jnp.where(counts > 0, last_tile - first_tile, 0)
    cum = jnp.cumsum(npairs)
    total = cum[-1]
    pair_start = cum - npairs
    wids = jnp.arange(W, dtype=jnp.int32)
    gid = jnp.searchsorted(cum.astype(jnp.int32), wids, side="right")
    gid = jnp.minimum(gid, E - 1).astype(jnp.int32)
    tid = (first_tile[gid] + (wids - pair_start[gid])).astype(jnp.int32)
    valid = wids < total
    tid = jnp.where(valid, tid, nt - 1).astype(jnp.int32)
    ws = jnp.where(valid, starts[gid], T).astype(jnp.int32)
    we = jnp.where(valid, ends[gid], T).astype(jnp.int32)
    fv = jnp.concatenate(
        [jnp.ones((1,), jnp.int32),
         (tid[1:] != tid[:-1]).astype(jnp.int32)])
    return perm, gid, tid, ws, we, fv


def _sc_gather_rows(table, idx):
    Tr, Cc = table.shape
    info = plsc.get_sparse_core_info()
    nw = info.num_cores * info.num_subcores
    bpw = Tr // nw
    mesh = plsc.VectorSubcoreMesh(core_axis_name="c", subcore_axis_name="s")

    @functools.partial(
        pl.kernel,
        mesh=mesh,
        out_type=jax.ShapeDtypeStruct((Tr, Cc), table.dtype),
        scratch_types=[
            pltpu.VMEM((bpw,), jnp.int32),
            pltpu.VMEM((bpw, Cc), table.dtype),
            pltpu.SemaphoreType.DMA,
        ],
    )
    def k(table_hbm, idx_hbm, out_hbm, idx_v, rows_v, sem):
        wid = lax.axis_index("s") * info.num_cores + lax.axis_index("c")
        base = wid * bpw
        pltpu.sync_copy(idx_hbm.at[pl.ds(base, bpw)], idx_v)
        pltpu.async_copy(table_hbm.at[idx_v], rows_v, sem).wait()
        pltpu.sync_copy(rows_v, out_hbm.at[pl.ds(base, bpw)])

    return k(table, idx)


def _ffn_body(gid_ref, tid_ref, ws_ref, we_ref, fv_ref,
              xs_ref, wi_ref, bi_ref, wg_ref, bg_ref, wo_ref, bo_ref,
              out_ref, acc_ref):
    h = pl.program_id(0)
    w = pl.program_id(1)
    nh = pl.num_programs(0)
    g = gid_ref[w]
    t = tid_ref[w]
    rows = t * TM + lax.broadcasted_iota(jnp.int32, (TM, 1), 0)
    mask = (rows >= ws_ref[w]) & (rows < we_ref[w])
    x = xs_ref[pl.ds(t * TM, TM), :]
    proj = jnp.dot(x, wi_ref[0], preferred_element_type=jnp.float32,
                   precision=lax.Precision.DEFAULT)
    proj = proj + bi_ref[pl.ds(g, 1), pl.ds(h * HK, HK)]
    gate = jnp.dot(x, wg_ref[0], preferred_element_type=jnp.float32,
                   precision=lax.Precision.DEFAULT)
    gate = gate + bg_ref[pl.ds(g, 1), pl.ds(h * HK, HK)]
    u = gate * lax.logistic(gate) * proj
    um = jnp.where(mask, u, 0.0).astype(jnp.bfloat16)
    part = jnp.dot(um, wo_ref[0].astype(jnp.bfloat16),
                   preferred_element_type=jnp.float32,
                   precision=lax.Precision.DEFAULT)
    sl = pl.ds(t * TM, TM)
    is_first = (h == 0) & (fv_ref[w] == 1)

    @pl.when(is_first)
    def _():
        acc_ref[sl, :] = part

    @pl.when(jnp.logical_not(is_first))
    def _():
        acc_ref[sl, :] += part

    @pl.when(h == nh - 1)
    def _():
        y = acc_ref[sl, :] + bo_ref[pl.ds(g, 1), :]
        out_ref[...] = jnp.where(mask, y, out_ref[...])


def _grouped_ffn(xs, Wi, bi, Wg, bg, Wo, bo, gid, tid, ws, we, fv):
    T, C = xs.shape
    E, _, H = Wi.shape
    W = gid.shape[0]
    nh = H // HK

    ys = pl.pallas_call(
        _ffn_body,
        grid_spec=pltpu.PrefetchScalarGridSpec(
            num_scalar_prefetch=5,
            grid=(nh, W),
            in_specs=[
                pl.BlockSpec((T, C), lambda h, w, *_: (0, 0)),
                pl.BlockSpec((1, C, HK), lambda h, w, gid, tid, ws, we, fv: (gid[w], 0, h)),
                pl.BlockSpec((E, H), lambda h, w, *_: (0, 0)),
                pl.BlockSpec((1, C, HK), lambda h, w, gid, tid, ws, we, fv: (gid[w], 0, h)),
                pl.BlockSpec((E, H), lambda h, w, *_: (0, 0)),
                pl.BlockSpec((1, HK, C), lambda h, w, gid, tid, ws, we, fv: (gid[w], h, 0)),
                pl.BlockSpec((E, C), lambda h, w, *_: (0, 0)),
            ],
            out_specs=pl.BlockSpec(
                (TM, C),
                lambda h, w, gid, tid, ws, we, fv: (
                    jnp.where(h == H // HK - 1, tid[w], 0), 0)),
            scratch_shapes=[pltpu.VMEM((T, C), jnp.float32)],
        ),
        out_shape=jax.ShapeDtypeStruct((T, C), jnp.float32),
        compiler_params=pltpu.CompilerParams(
            dimension_semantics=("arbitrary", "arbitrary")),
    )(gid, tid, ws, we, fv, xs, Wi, bi, Wg, bg, Wo, bo)
    return ys


def kernel(x, Wi, bi, Wg, bg, Wo, bo, rand_maps):
    B, T_, C = x.shape
    E = Wi.shape[0]
    T = B * T_
    xf = x.reshape(T, C)

    perm, gid, tid, ws, we, fv = _routing_metadata(rand_maps, T, E, TM)
    inv_perm = jnp.argsort(perm).astype(jnp.int32)

    xs = _sc_gather_rows(xf, perm)
    ys = _grouped_ffn(xs, Wi, bi, Wg, bg, Wo, bo, gid, tid, ws, we, fv)
    outf = _sc_gather_rows(ys, inv_perm)
    return outf.reshape(B, T_, C)


# speedup vs baseline: 1.1437x; 1.1437x over previous
import functools

import jax
import jax.numpy as jnp
from jax import lax
from jax.experimental import pallas as pl
from jax.experimental.pallas import tpu as pltpu
from jax.experimental.pallas import tpu_sc as plsc

TM = 256
HK = 1024


def _routing_metadata(rand_maps, T, E, tm):
    nt = T // tm
    W = nt + E - 1
    perm = jnp.argsort(rand_maps, stable=True).astype(jnp.int32)
    counts = jnp.bincount(rand_maps, length=E)
    ends = jnp.cumsum(counts)
    starts = ends - counts
    first_tile = starts // tm
    last_tile = (ends + tm - 1) // tm
    npairs = jnp.where(counts > 0, last_tile - first_tile, 0)
    cum = jnp.cumsum(npairs)
    total = cum[-1]
    pair_start = cum - npairs
    wids = jnp.arange(W, dtype=jnp.int32)
    gid = jnp.searchsorted(cum.astype(jnp.int32), wids, side="right")
    gid = jnp.minimum(gid, E - 1).astype(jnp.int32)
    tid = (first_tile[gid] + (wids - pair_start[gid])).astype(jnp.int32)
    valid = wids < total
    tid = jnp.where(valid, tid, nt - 1).astype(jnp.int32)
    ws = jnp.where(valid, starts[gid], T).astype(jnp.int32)
    we = jnp.where(valid, ends[gid], T).astype(jnp.int32)
    fv = jnp.concatenate(
        [jnp.ones((1,), jnp.int32),
         (tid[1:] != tid[:-1]).astype(jnp.int32)])
    return perm, gid, tid, ws, we, fv


def _sc_gather_rows(table, idx):
    Tr, Cc = table.shape
    info = plsc.get_sparse_core_info()
    nw = info.num_cores * info.num_subcores
    bpw = Tr // nw
    mesh = plsc.VectorSubcoreMesh(core_axis_name="c", subcore_axis_name="s")

    @functools.partial(
        pl.kernel,
        mesh=mesh,
        out_type=jax.ShapeDtypeStruct((Tr, Cc), table.dtype),
        scratch_types=[
            pltpu.VMEM((bpw,), jnp.int32),
            pltpu.VMEM((bpw, Cc), table.dtype),
            pltpu.SemaphoreType.DMA,
        ],
    )
    def k(table_hbm, idx_hbm, out_hbm, idx_v, rows_v, sem):
        wid = lax.axis_index("s") * info.num_cores + lax.axis_index("c")
        base = wid * bpw
        pltpu.sync_copy(idx_hbm.at[pl.ds(base, bpw)], idx_v)
        pltpu.async_copy(table_hbm.at[idx_v], rows_v, sem).wait()
        pltpu.sync_copy(rows_v, out_hbm.at[pl.ds(base, bpw)])

    return k(table, idx)


def _sc_scatter_rows(rows, idx):
    Tr, Cc = rows.shape
    info = plsc.get_sparse_core_info()
    nw = info.num_cores * info.num_subcores
    bpw = Tr // nw
    mesh = plsc.VectorSubcoreMesh(core_axis_name="c", subcore_axis_name="s")

    @functools.partial(
        pl.kernel,
        mesh=mesh,
        out_type=jax.ShapeDtypeStruct((Tr, Cc), rows.dtype),
        scratch_types=[
            pltpu.VMEM((bpw,), jnp.int32),
            pltpu.VMEM((bpw, Cc), rows.dtype),
            pltpu.SemaphoreType.DMA,
        ],
    )
    def k(rows_hbm, idx_hbm, out_hbm, idx_v, rows_v, sem):
        wid = lax.axis_index("s") * info.num_cores + lax.axis_index("c")
        base = wid * bpw
        pltpu.sync_copy(idx_hbm.at[pl.ds(base, bpw)], idx_v)
        pltpu.sync_copy(rows_hbm.at[pl.ds(base, bpw)], rows_v)
        pltpu.async_copy(rows_v, out_hbm.at[idx_v], sem).wait()

    return k(rows, idx)


def _ffn_body(gid_ref, tid_ref, ws_ref, we_ref, fv_ref,
              xs_ref, wi_ref, bi_ref, wg_ref, bg_ref, wo_ref, bo_ref,
              out_ref, acc_ref):
    h = pl.program_id(0)
    w = pl.program_id(1)
    nh = pl.num_programs(0)
    g = gid_ref[w]
    t = tid_ref[w]
    rows = t * TM + lax.broadcasted_iota(jnp.int32, (TM, 1), 0)
    mask = (rows >= ws_ref[w]) & (rows < we_ref[w])
    x = xs_ref[pl.ds(t * TM, TM), :]
    proj = jnp.dot(x, wi_ref[0], preferred_element_type=jnp.float32,
                   precision=lax.Precision.DEFAULT)
    proj = proj + bi_ref[pl.ds(g, 1), pl.ds(h * HK, HK)]
    gate = jnp.dot(x, wg_ref[0], preferred_element_type=jnp.float32,
                   precision=lax.Precision.DEFAULT)
    gate = gate + bg_ref[pl.ds(g, 1), pl.ds(h * HK, HK)]
    u = gate * lax.logistic(gate) * proj
    um = jnp.where(mask, u, 0.0).astype(jnp.bfloat16)
    part = jnp.dot(um, wo_ref[0].astype(jnp.bfloat16),
                   preferred_element_type=jnp.float32,
                   precision=lax.Precision.DEFAULT)
    sl = pl.ds(t * TM, TM)
    is_first = (h == 0) & (fv_ref[w] == 1)

    @pl.when(is_first)
    def _():
        acc_ref[sl, :] = part

    @pl.when(jnp.logical_not(is_first))
    def _():
        acc_ref[sl, :] += part

    @pl.when(h == nh - 1)
    def _():
        y = acc_ref[sl, :] + bo_ref[pl.ds(g, 1), :]
        out_ref[...] = jnp.where(mask, y, out_ref[...])


def _grouped_ffn(xs, Wi, bi, Wg, bg, Wo, bo, gid, tid, ws, we, fv):
    T, C = xs.shape
    E, _, H = Wi.shape
    W = gid.shape[0]
    nh = H // HK

    ys = pl.pallas_call(
        _ffn_body,
        grid_spec=pltpu.PrefetchScalarGridSpec(
            num_scalar_prefetch=5,
            grid=(nh, W),
            in_specs=[
                pl.BlockSpec((T, C), lambda h, w, *_: (0, 0)),
                pl.BlockSpec((1, C, HK), lambda h, w, gid, tid, ws, we, fv: (gid[w], 0, h)),
                pl.BlockSpec((E, H), lambda h, w, *_: (0, 0)),
                pl.BlockSpec((1, C, HK), lambda h, w, gid, tid, ws, we, fv: (gid[w], 0, h)),
                pl.BlockSpec((E, H), lambda h, w, *_: (0, 0)),
                pl.BlockSpec((1, HK, C), lambda h, w, gid, tid, ws, we, fv: (gid[w], h, 0)),
                pl.BlockSpec((E, C), lambda h, w, *_: (0, 0)),
            ],
            out_specs=pl.BlockSpec(
                (TM, C),
                lambda h, w, gid, tid, ws, we, fv: (
                    jnp.where(h == H // HK - 1, tid[w], 0), 0)),
            scratch_shapes=[pltpu.VMEM((T, C), jnp.float32)],
        ),
        out_shape=jax.ShapeDtypeStruct((T, C), jnp.float32),
        compiler_params=pltpu.CompilerParams(
            dimension_semantics=("arbitrary", "arbitrary")),
    )(gid, tid, ws, we, fv, xs, Wi, bi, Wg, bg, Wo, bo)
    return ys


def kernel(x, Wi, bi, Wg, bg, Wo, bo, rand_maps):
    B, T_, C = x.shape
    E = Wi.shape[0]
    T = B * T_
    xf = x.reshape(T, C)

    perm, gid, tid, ws, we, fv = _routing_metadata(rand_maps, T, E, TM)

    xs = _sc_gather_rows(xf, perm)
    ys = _grouped_ffn(xs, Wi, bi, Wg, bg, Wo, bo, gid, tid, ws, we, fv)
    outf = _sc_scatter_rows(ys, perm)
    return outf.reshape(B, T_, C)


# speedup vs baseline: 1.3325x; 1.1651x over previous
import functools

import jax
import jax.numpy as jnp
from jax import lax
from jax.experimental import pallas as pl
from jax.experimental.pallas import tpu as pltpu
from jax.experimental.pallas import tpu_sc as plsc

TM = 256
HK = 1024


def _routing_metadata(rand_maps, T, E, tm):
    perm = jnp.argsort(rand_maps, stable=True).astype(jnp.int32)
    counts = jnp.bincount(rand_maps, length=E)
    ends = jnp.cumsum(counts).astype(jnp.int32)
    starts = (ends - counts).astype(jnp.int32)
    first_tile = (starts // tm).astype(jnp.int32)
    ntiles = jnp.where(counts > 0,
                       (ends + tm - 1) // tm - first_tile, 0).astype(jnp.int32)
    return perm, first_tile, ntiles, starts, ends


def _sc_gather_rows(table, idx):
    Tr, Cc = table.shape
    info = plsc.get_sparse_core_info()
    nw = info.num_cores * info.num_subcores
    bpw = Tr // nw
    mesh = plsc.VectorSubcoreMesh(core_axis_name="c", subcore_axis_name="s")

    @functools.partial(
        pl.kernel,
        mesh=mesh,
        out_type=jax.ShapeDtypeStruct((Tr, Cc), table.dtype),
        scratch_types=[
            pltpu.VMEM((bpw,), jnp.int32),
            pltpu.VMEM((bpw, Cc), table.dtype),
            pltpu.SemaphoreType.DMA,
        ],
    )
    def k(table_hbm, idx_hbm, out_hbm, idx_v, rows_v, sem):
        wid = lax.axis_index("s") * info.num_cores + lax.axis_index("c")
        base = wid * bpw
        pltpu.sync_copy(idx_hbm.at[pl.ds(base, bpw)], idx_v)
        pltpu.async_copy(table_hbm.at[idx_v], rows_v, sem).wait()
        pltpu.sync_copy(rows_v, out_hbm.at[pl.ds(base, bpw)])

    return k(table, idx)


def _sc_scatter_rows(rows, idx):
    Tr, Cc = rows.shape
    info = plsc.get_sparse_core_info()
    nw = info.num_cores * info.num_subcores
    bpw = Tr // nw
    mesh = plsc.VectorSubcoreMesh(core_axis_name="c", subcore_axis_name="s")

    @functools.partial(
        pl.kernel,
        mesh=mesh,
        out_type=jax.ShapeDtypeStruct((Tr, Cc), rows.dtype),
        scratch_types=[
            pltpu.VMEM((bpw,), jnp.int32),
            pltpu.VMEM((bpw, Cc), rows.dtype),
            pltpu.SemaphoreType.DMA,
        ],
    )
    def k(rows_hbm, idx_hbm, out_hbm, idx_v, rows_v, sem):
        wid = lax.axis_index("s") * info.num_cores + lax.axis_index("c")
        base = wid * bpw
        pltpu.sync_copy(idx_hbm.at[pl.ds(base, bpw)], idx_v)
        pltpu.sync_copy(rows_hbm.at[pl.ds(base, bpw)], rows_v)
        pltpu.async_copy(rows_v, out_hbm.at[idx_v], sem).wait()

    return k(rows, idx)


def _ffn_body(ft_ref, nt_ref, ws_ref, we_ref,
              xs_ref, wi_ref, bi_ref, wg_ref, bg_ref, wo_ref, bo_ref,
              out_ref):
    h = pl.program_id(0)
    g = pl.program_id(1)
    nh = pl.num_programs(0)
    ws = ws_ref[g]
    we = we_ref[g]
    ft = ft_ref[g]
    wi = wi_ref[0]
    wg = wg_ref[0]
    wo = wo_ref[0].astype(jnp.bfloat16)

    def tile_step(i, carry):
        t = ft + i
        rows = t * TM + lax.broadcasted_iota(jnp.int32, (TM, 1), 0)
        mask = (rows >= ws) & (rows < we)
        x = xs_ref[pl.ds(t * TM, TM), :]
        proj = jnp.dot(x, wi, preferred_element_type=jnp.float32,
                       precision=lax.Precision.DEFAULT)
        proj = proj + bi_ref[pl.ds(g, 1), pl.ds(h * HK, HK)]
        gate = jnp.dot(x, wg, preferred_element_type=jnp.float32,
                       precision=lax.Precision.DEFAULT)
        gate = gate + bg_ref[pl.ds(g, 1), pl.ds(h * HK, HK)]
        u = gate * lax.logistic(gate) * proj
        um = jnp.where(mask, u, 0.0).astype(jnp.bfloat16)
        part = jnp.dot(um, wo, preferred_element_type=jnp.float32,
                       precision=lax.Precision.DEFAULT)
        sl = pl.ds(t * TM, TM)
        prev = out_ref[sl, :]

        @pl.when(h == 0)
        def _():
            out_ref[sl, :] = jnp.where(mask, part, prev)

        @pl.when(jnp.logical_and(h > 0, h < nh - 1))
        def _():
            out_ref[sl, :] = jnp.where(mask, prev + part, prev)

        @pl.when(h == nh - 1)
        def _():
            y = prev + part + bo_ref[pl.ds(g, 1), :]
            out_ref[sl, :] = jnp.where(mask, y, prev)

        return carry

    lax.fori_loop(0, nt_ref[g], tile_step, 0)


def _grouped_ffn(xs, Wi, bi, Wg, bg, Wo, bo, ft, ntl, ws, we):
    T, C = xs.shape
    E, _, H = Wi.shape
    nh = H // HK

    ys = pl.pallas_call(
        _ffn_body,
        grid_spec=pltpu.PrefetchScalarGridSpec(
            num_scalar_prefetch=4,
            grid=(nh, E),
            in_specs=[
                pl.BlockSpec((T, C), lambda h, g, *_: (0, 0)),
                pl.BlockSpec((1, C, HK), lambda h, g, *_: (g, 0, h)),
                pl.BlockSpec((E, H), lambda h, g, *_: (0, 0)),
                pl.BlockSpec((1, C, HK), lambda h, g, *_: (g, 0, h)),
                pl.BlockSpec((E, H), lambda h, g, *_: (0, 0)),
                pl.BlockSpec((1, HK, C), lambda h, g, *_: (g, h, 0)),
                pl.BlockSpec((E, C), lambda h, g, *_: (0, 0)),
            ],
            out_specs=pl.BlockSpec((T, C), lambda h, g, *_: (0, 0)),
        ),
        out_shape=jax.ShapeDtypeStruct((T, C), jnp.float32),
        compiler_params=pltpu.CompilerParams(
            dimension_semantics=("arbitrary", "arbitrary")),
    )(ft, ntl, ws, we, xs, Wi, bi, Wg, bg, Wo, bo)
    return ys


def kernel(x, Wi, bi, Wg, bg, Wo, bo, rand_maps):
    B, T_, C = x.shape
    E = Wi.shape[0]
    T = B * T_
    xf = x.reshape(T, C)

    perm, ft, ntl, ws, we = _routing_metadata(rand_maps, T, E, TM)

    xs = _sc_gather_rows(xf, perm)
    ys = _grouped_ffn(xs, Wi, bi, Wg, bg, Wo, bo, ft, ntl, ws, we)
    outf = _sc_scatter_rows(ys, perm)
    return outf.reshape(B, T_, C)
